# baseline (speedup 1.0000x reference)
import math

import jax
import jax.numpy as jnp
from jax.experimental import pallas as pl
from jax.experimental.pallas import tpu as pltpu

K = 1024
D = 32
LMBDA = 0.05


def _vq_body(x_ref, sx2_ref, et_ref, e_ref, e2_ref, bias_ref, l2p_ref,
             q_ref, ind_ref, stats_ref):
    n = pl.program_id(0)
    hb = pl.program_id(1)
    W = x_ref.shape[2]
    xb = x_ref[0]
    mm = jnp.dot(e_ref[...], xb, preferred_element_type=jnp.float32)
    dist = ((sx2_ref[0] + e2_ref[...]) - 2.0 * mm) + bias_ref[...]
    dmin = jnp.min(dist, axis=0, keepdims=True)
    kio = jax.lax.broadcasted_iota(jnp.int32, (K, W), 0)
    ind = jnp.min(jnp.where(dist == dmin, kio, K), axis=0, keepdims=True)
    oh = (kio == ind).astype(jnp.float32)
    qt = jnp.dot(et_ref[...], oh, preferred_element_type=jnp.float32)
    q_ref[...] = qt[None]
    ind_ref[...] = ind[None]
    mse_p = jnp.sum((qt - xb) ** 2)
    rate_p = jnp.sum(oh * l2p_ref[...])
    sio = jax.lax.broadcasted_iota(jnp.int32, (8, 128), 0)
    upd = (jnp.where(sio == 0, mse_p, 0.0)
           + jnp.where(sio == 1, rate_p, 0.0))

    @pl.when((n == 0) & (hb == 0))
    def _():
        stats_ref[...] = jnp.zeros_like(stats_ref)

    stats_ref[...] += upd


def kernel(latents, embedding_weight, pmf_logits):
    N, H, W = latents.shape
    target_rows = H % D
    if target_rows != 0:
        pad_len = D - target_rows
        latents_e = jnp.concatenate([latents, latents[:, -pad_len:, :]], axis=1)
    else:
        latents_e = latents
    Hp = latents_e.shape[1]
    HB = Hp // D
    M = N * W * HB

    flat = jnp.transpose(latents_e, (0, 2, 1)).reshape(N, W, HB, D).reshape(-1, D)
    sx2 = jnp.sum(flat ** 2, axis=1)
    sx2_g = jnp.transpose(sx2.reshape(N, W, HB), (0, 2, 1)).reshape(N * HB, 1, W)
    e2 = jnp.sum(embedding_weight ** 2, axis=1)[:, None]
    log_pmf = jax.nn.log_softmax(pmf_logits)
    log2_pmf = log_pmf / -math.log(2.0)
    rate_bias = (log2_pmf / LMBDA)[:, None]
    l2p = log2_pmf[:, None]
    et = embedding_weight.T

    qe, inds_g, stats = pl.pallas_call(
        _vq_body,
        grid=(N, HB),
        in_specs=[
            pl.BlockSpec((1, D, W), lambda n, hb: (n, hb, 0)),
            pl.BlockSpec((1, 1, W), lambda n, hb: (n * HB + hb, 0, 0)),
            pl.BlockSpec((D, K), lambda n, hb: (0, 0)),
            pl.BlockSpec((K, D), lambda n, hb: (0, 0)),
            pl.BlockSpec((K, 1), lambda n, hb: (0, 0)),
            pl.BlockSpec((K, 1), lambda n, hb: (0, 0)),
            pl.BlockSpec((K, 1), lambda n, hb: (0, 0)),
        ],
        out_specs=[
            pl.BlockSpec((1, D, W), lambda n, hb: (n, hb, 0)),
            pl.BlockSpec((1, 1, W), lambda n, hb: (n * HB + hb, 0, 0)),
            pl.BlockSpec((8, 128), lambda n, hb: (0, 0)),
        ],
        out_shape=[
            jax.ShapeDtypeStruct((N, Hp, W), jnp.float32),
            jax.ShapeDtypeStruct((N * HB, 1, W), jnp.int32),
            jax.ShapeDtypeStruct((8, 128), jnp.float32),
        ],
    )(latents_e, sx2_g, et, embedding_weight, e2, rate_bias, l2p)

    quantized = qe[:, :H, :]
    inds = jnp.transpose(inds_g.reshape(N, HB, W), (0, 2, 1)).reshape(M, 1)
    mse_loss = stats[0, 0] / jnp.float32(M * D)
    rate_uem = stats[1, 0]
    prior_dist = jnp.zeros(1, dtype=jnp.float32)
    param_bit = jnp.zeros(1, dtype=jnp.float32)
    return (quantized, mse_loss, inds, rate_uem, prior_dist, param_bit)


# speedup vs baseline: 1.3032x; 1.3032x over previous
import math

import jax
import jax.numpy as jnp
from jax.experimental import pallas as pl
from jax.experimental.pallas import tpu as pltpu

K = 1024
D = 32
LMBDA = 0.05
DA = D + 4


def _vq_body(x_ref, sx2_ref, e2x_ref, eaug_ref, e2_ref, bias_ref,
             q_ref, ind_ref, stats_ref):
    n = pl.program_id(0)
    hb = pl.program_id(1)
    W = x_ref.shape[2]
    xb = x_ref[0]
    mm2 = jnp.dot(e2x_ref[...], xb, preferred_element_type=jnp.float32)
    dist = ((sx2_ref[0] + e2_ref[...]) - mm2) + bias_ref[...]
    dmin = jnp.min(dist, axis=0, keepdims=True)
    eq = dist == dmin
    ohm = eq.astype(jnp.float32)
    aug = jnp.dot(eaug_ref[...], ohm, preferred_element_type=jnp.float32)
    ties = jnp.max(aug[DA - 1:DA, :]) > 1.5

    @pl.when((n == 0) & (hb == 0))
    def _():
        stats_ref[...] = jnp.zeros_like(stats_ref)

    def _finish(a, ind):
        qt = a[:D]
        q_ref[...] = qt[None]
        ind_ref[...] = ind[None]
        mse_p = jnp.sum((qt - xb) ** 2)
        rate_p = jnp.sum(a[D + 2:D + 3, :])
        sio = jax.lax.broadcasted_iota(jnp.int32, (8, 128), 0)
        stats_ref[...] += (jnp.where(sio == 0, mse_p, 0.0)
                           + jnp.where(sio == 1, rate_p, 0.0))

    @pl.when(jnp.logical_not(ties))
    def _():
        ind = (aug[D:D + 1, :] * 256.0 + aug[D + 1:D + 2, :]).astype(jnp.int32)
        _finish(aug, ind)

    @pl.when(ties)
    def _():
        kio = jax.lax.broadcasted_iota(jnp.int32, (K, W), 0)
        ind = jnp.min(jnp.where(eq, kio, K), axis=0, keepdims=True)
        oh = (kio == ind).astype(jnp.float32)
        aug2 = jnp.dot(eaug_ref[...], oh, preferred_element_type=jnp.float32)
        _finish(aug2, ind)


def kernel(latents, embedding_weight, pmf_logits):
    N, H, W = latents.shape
    target_rows = H % D
    if target_rows != 0:
        pad_len = D - target_rows
        latents_e = jnp.concatenate([latents, latents[:, -pad_len:, :]], axis=1)
    else:
        latents_e = latents
    Hp = latents_e.shape[1]
    HB = Hp // D
    M = N * W * HB

    sx2_g = jnp.sum(latents_e.reshape(N, HB, D, W) ** 2,
                    axis=2).reshape(N * HB, 1, W)
    e2 = jnp.sum(embedding_weight ** 2, axis=1)[:, None]
    log_pmf = jax.nn.log_softmax(pmf_logits)
    log2_pmf = log_pmf / -math.log(2.0)
    rate_bias = (log2_pmf / LMBDA)[:, None]
    kk = jnp.arange(K, dtype=jnp.int32)
    eaug = jnp.concatenate([
        embedding_weight.T,
        (kk // 256).astype(jnp.float32)[None, :],
        (kk % 256).astype(jnp.float32)[None, :],
        log2_pmf[None, :],
        jnp.ones((1, K), jnp.float32),
    ], axis=0)

    qe, inds_g, stats = pl.pallas_call(
        _vq_body,
        grid=(N, HB),
        in_specs=[
            pl.BlockSpec((1, D, W), lambda n, hb: (n, hb, 0)),
            pl.BlockSpec((1, 1, W), lambda n, hb: (n * HB + hb, 0, 0)),
            pl.BlockSpec((K, D), lambda n, hb: (0, 0)),
            pl.BlockSpec((DA, K), lambda n, hb: (0, 0)),
            pl.BlockSpec((K, 1), lambda n, hb: (0, 0)),
            pl.BlockSpec((K, 1), lambda n, hb: (0, 0)),
        ],
        out_specs=[
            pl.BlockSpec((1, D, W), lambda n, hb: (n, hb, 0)),
            pl.BlockSpec((1, 1, W), lambda n, hb: (n * HB + hb, 0, 0)),
            pl.BlockSpec((8, 128), lambda n, hb: (0, 0)),
        ],
        out_shape=[
            jax.ShapeDtypeStruct((N, Hp, W), jnp.float32),
            jax.ShapeDtypeStruct((N * HB, 1, W), jnp.int32),
            jax.ShapeDtypeStruct((8, 128), jnp.float32),
        ],
    )(latents_e, sx2_g, embedding_weight * 2.0, eaug, e2, rate_bias)

    quantized = qe[:, :H, :]
    inds = jnp.transpose(inds_g.reshape(N, HB, W), (0, 2, 1)).reshape(M, 1)
    mse_loss = stats[0, 0] / jnp.float32(M * D)
    rate_uem = stats[1, 0]
    prior_dist = jnp.zeros(1, dtype=jnp.float32)
    param_bit = jnp.zeros(1, dtype=jnp.float32)
    return (quantized, mse_loss, inds, rate_uem, prior_dist, param_bit)
